# baseline (speedup 1.0000x reference)
import functools

import jax
import jax.numpy as jnp
from jax import lax
from jax.experimental import pallas as pl
from jax.experimental.pallas import tpu as pltpu
from jax.experimental.pallas import tpu_sc as plsc

B, C_IN, N = 8, 64, 16384
NS, K = 4096, 16
C_OUT = 128
L = 16

NC, NSUB = 2, 16
NW = NC * NSUB
CH = (B * NS) // NW
WPB = NW // B
SUBC = 256
NSUBCH = CH // SUBC
GRP = 8
NGRP = SUBC // GRP
NBUF = 4


def _z_body(x_ref, p_ref, w1_ref, w2_ref, o_ref):
    f = jnp.concatenate([x_ref[0], p_ref[0]], axis=0)
    w = jnp.concatenate([w1_ref[...], w2_ref[...]], axis=0)
    z = lax.dot_general(f, w, (((0,), (0,)), ((), ())),
                        preferred_element_type=jnp.float32)
    o_ref[...] = z.astype(jnp.bfloat16)


def _s_body(sp_ref, w2_ref, b_ref, o_ref):
    s = lax.dot_general(sp_ref[0], w2_ref[...], (((0,), (0,)), ((), ())),
                        preferred_element_type=jnp.float32)
    o_ref[...] = (s - b_ref[...]).astype(jnp.bfloat16)


def _sc_body(z_hbm, idx_hbm, s_hbm, out_hbm,
             idx_v, s_v, b0, b1, b2, b3, out_t, m0, m1, m2, m3):
    bufs = (b0, b1, b2, b3)
    sems = (m0, m1, m2, m3)
    cid = lax.axis_index("c")
    sid = lax.axis_index("s")
    wid = sid * NC + cid
    b = wid // WPB
    nbase = (wid % WPB) * CH

    def start_gather(g, buf, sem):
        off = pl.multiple_of(g * (GRP * K), GRP * K)
        pltpu.make_async_copy(
            z_hbm.at[idx_v.at[pl.ds(off, GRP * K)]], buf, sem).start()

    def wait_gather(g, buf, sem):
        off = pl.multiple_of(g * (GRP * K), GRP * K)
        pltpu.make_async_copy(
            z_hbm.at[idx_v.at[pl.ds(off, GRP * K)]], buf, sem).wait()

    def compute_group(g, buf):
        pp0 = g * GRP
        iota2 = lax.iota(jnp.int32, L) * 2
        himask = jnp.full((L,), -65536, jnp.int32)
        for p in range(GRP):
            pp = pp0 + p
            cols = jnp.full((L,), pp, jnp.int32)
            for q in range(C_OUT // (2 * L)):
                sl = pl.ds(2 * L * q, 2 * L)
                vals = [buf[K * p + r, sl] for r in range(K)]
                while len(vals) > 1:
                    vals = [jnp.maximum(vals[2 * i], vals[2 * i + 1])
                            for i in range(len(vals) // 2)]
                mi = plsc.bitcast(vals[0], jnp.int32)
                lo = plsc.bitcast(mi << 16, jnp.float32)
                hi = plsc.bitcast(mi & himask, jnp.float32)
                svi = plsc.bitcast(s_v[pp, sl], jnp.int32)
                s_lo = plsc.bitcast(svi << 16, jnp.float32)
                s_hi = plsc.bitcast(svi & himask, jnp.float32)
                rows_lo = iota2 + (2 * L * q)
                plsc.store_scatter(out_t, [rows_lo, cols], lo - s_lo)
                plsc.store_scatter(out_t, [rows_lo + 1, cols], hi - s_hi)

    def sub_body(s_i, carry):
        base = pl.multiple_of(wid * CH + s_i * SUBC, SUBC)
        pltpu.sync_copy(idx_hbm.at[pl.ds(base * K, SUBC * K)], idx_v)
        pltpu.sync_copy(s_hbm.at[pl.ds(base, SUBC)], s_v)

        boff = b * N

        def shift_body(i, c):
            sl = pl.ds(pl.multiple_of(i * L, L), L)
            idx_v[sl] = idx_v[sl] + boff
            return c
        lax.fori_loop(0, (SUBC * K) // L, shift_body, 0)

        for j in range(NBUF):
            start_gather(j, bufs[j], sems[j])

        def grp_body(i, c):
            g0 = NBUF * i
            for j in range(NBUF):
                g = g0 + j
                wait_gather(g, bufs[j], sems[j])
                compute_group(g, bufs[j])

                @pl.when(g + NBUF < NGRP)
                def _():
                    start_gather(g + NBUF, bufs[j], sems[j])
            return c
        lax.fori_loop(0, NGRP // NBUF, grp_body, 0)

        n0 = pl.multiple_of(nbase + s_i * SUBC, SUBC)
        pltpu.sync_copy(out_t, out_hbm.at[b, :, pl.ds(n0, SUBC)])
        return carry
    lax.fori_loop(0, NSUBCH, sub_body, 0)


def kernel(input, points, support_points, indices, W, bbias):
    w1 = W[:C_IN]
    w2 = W[C_IN:]
    TN = 2048

    zflat = pl.pallas_call(
        _z_body,
        grid=(B, N // TN),
        in_specs=[
            pl.BlockSpec((1, C_IN, TN), lambda b, t: (b, 0, t)),
            pl.BlockSpec((1, 3, TN), lambda b, t: (b, 0, t)),
            pl.BlockSpec((C_IN, C_OUT), lambda b, t: (0, 0)),
            pl.BlockSpec((3, C_OUT), lambda b, t: (0, 0)),
        ],
        out_specs=pl.BlockSpec((TN, C_OUT),
                               lambda b, t: (b * (N // TN) + t, 0)),
        out_shape=jax.ShapeDtypeStruct((B * N, C_OUT), jnp.bfloat16),
    )(input, points, w1, w2)

    sflat = pl.pallas_call(
        _s_body,
        grid=(B,),
        in_specs=[
            pl.BlockSpec((1, 3, NS), lambda b: (b, 0, 0)),
            pl.BlockSpec((3, C_OUT), lambda b: (0, 0)),
            pl.BlockSpec((1, C_OUT), lambda b: (0, 0)),
        ],
        out_specs=pl.BlockSpec((NS, C_OUT), lambda b: (b, 0)),
        out_shape=jax.ShapeDtypeStruct((B * NS, C_OUT), jnp.bfloat16),
    )(support_points, w2, bbias.reshape(1, C_OUT))

    idx_flat = indices.astype(jnp.int32).reshape(-1)

    mesh = plsc.VectorSubcoreMesh(core_axis_name="c", subcore_axis_name="s",
                                  num_cores=NC, num_subcores=NSUB)
    out = pl.kernel(
        _sc_body,
        out_type=jax.ShapeDtypeStruct((B, C_OUT, NS), jnp.float32),
        mesh=mesh,
        compiler_params=pltpu.CompilerParams(use_tc_tiling_on_sc=False,
                                             needs_layout_passes=False),
        scratch_types=[
            pltpu.VMEM((SUBC * K,), jnp.int32),
            pltpu.VMEM((SUBC, C_OUT), jnp.bfloat16),
            pltpu.VMEM((GRP * K, C_OUT), jnp.bfloat16),
            pltpu.VMEM((GRP * K, C_OUT), jnp.bfloat16),
            pltpu.VMEM((GRP * K, C_OUT), jnp.bfloat16),
            pltpu.VMEM((GRP * K, C_OUT), jnp.bfloat16),
            pltpu.VMEM((C_OUT, SUBC), jnp.float32),
            pltpu.SemaphoreType.DMA,
            pltpu.SemaphoreType.DMA,
            pltpu.SemaphoreType.DMA,
            pltpu.SemaphoreType.DMA,
        ],
    )(zflat, idx_flat, sflat)

    return (out, support_points, indices)


# speedup vs baseline: 1.2453x; 1.2453x over previous
import functools

import jax
import jax.numpy as jnp
from jax import lax
from jax.experimental import pallas as pl
from jax.experimental.pallas import tpu as pltpu
from jax.experimental.pallas import tpu_sc as plsc

B, C_IN, N = 8, 64, 16384
NS, K = 4096, 16
C_OUT = 128
L = 16

NC, NSUB = 2, 16
NW = NC * NSUB
CH = (B * NS) // NW
WPB = NW // B
SUBC = 256
NSUBCH = CH // SUBC
GRP = 8
NGRP = SUBC // GRP
NBUF = 2
CW = C_OUT // 2


def _pack_bf16_halves(z):
    zi = lax.bitcast_convert_type(z, jnp.int32)
    rb = (zi + 0x7FFF + ((zi >> 16) & 1)) >> 16
    return (rb[:, :CW] & 0xFFFF) | (rb[:, CW:] << 16)


def _z_body(x_ref, p_ref, w1_ref, w2_ref, o_ref):
    f = jnp.concatenate([x_ref[0], p_ref[0]], axis=0)
    w = jnp.concatenate([w1_ref[...], w2_ref[...]], axis=0)
    z = lax.dot_general(f, w, (((0,), (0,)), ((), ())),
                        preferred_element_type=jnp.float32)
    o_ref[...] = _pack_bf16_halves(z)


def _s_body(sp_ref, w2_ref, b_ref, o_ref):
    s = lax.dot_general(sp_ref[0], w2_ref[...], (((0,), (0,)), ((), ())),
                        preferred_element_type=jnp.float32)
    o_ref[...] = _pack_bf16_halves(s - b_ref[...])


def _sc_body(z_hbm, idx_hbm, s_hbm, out_hbm,
             idx_v, s_v, b0, b1, out_t, m0, m1):
    bufs = (b0, b1)
    sems = (m0, m1)
    cid = lax.axis_index("c")
    sid = lax.axis_index("s")
    wid = sid * NC + cid
    b = wid // WPB
    nbase = (wid % WPB) * CH

    def start_gather(g, buf, sem):
        off = pl.multiple_of(g * (GRP * K), GRP * K)
        pltpu.make_async_copy(
            z_hbm.at[idx_v.at[pl.ds(off, GRP * K)]], buf, sem).start()

    def wait_gather(g, buf, sem):
        off = pl.multiple_of(g * (GRP * K), GRP * K)
        pltpu.make_async_copy(
            z_hbm.at[idx_v.at[pl.ds(off, GRP * K)]], buf, sem).wait()

    def compute_group(g, buf):
        pp0 = g * GRP
        iota = lax.iota(jnp.int32, L)
        himask = jnp.full((L,), -65536, jnp.int32)
        for p in range(GRP):
            pp = pp0 + p
            cols = jnp.full((L,), pp, jnp.int32)
            for q in range(CW // L):
                sl = pl.ds(L * q, L)
                acc = plsc.bitcast(buf[K * p, sl], jnp.bfloat16)
                for r in range(1, K):
                    acc = jnp.maximum(
                        acc, plsc.bitcast(buf[K * p + r, sl], jnp.bfloat16))
                mi = plsc.bitcast(acc, jnp.int32)
                lo = plsc.bitcast(mi << 16, jnp.float32)
                hi = plsc.bitcast(mi & himask, jnp.float32)
                svi = s_v[pp, sl]
                s_lo = plsc.bitcast(svi << 16, jnp.float32)
                s_hi = plsc.bitcast(svi & himask, jnp.float32)
                rows_lo = iota + (L * q)
                plsc.store_scatter(out_t, [rows_lo, cols], lo - s_lo)
                plsc.store_scatter(out_t, [rows_lo + CW, cols], hi - s_hi)

    def sub_body(s_i, carry):
        base = pl.multiple_of(wid * CH + s_i * SUBC, SUBC)
        pltpu.sync_copy(idx_hbm.at[pl.ds(base * K, SUBC * K)], idx_v)
        pltpu.sync_copy(s_hbm.at[pl.ds(base, SUBC)], s_v)

        boff = b * N

        def shift_body(i, c):
            sl = pl.ds(pl.multiple_of(i * L, L), L)
            idx_v[sl] = idx_v[sl] + boff
            return c
        lax.fori_loop(0, (SUBC * K) // L, shift_body, 0)

        for j in range(NBUF):
            start_gather(j, bufs[j], sems[j])

        def grp_body(i, c):
            g0 = NBUF * i
            for j in range(NBUF):
                g = g0 + j
                wait_gather(g, bufs[j], sems[j])
                compute_group(g, bufs[j])

                @pl.when(g + NBUF < NGRP)
                def _():
                    start_gather(g + NBUF, bufs[j], sems[j])
            return c
        lax.fori_loop(0, NGRP // NBUF, grp_body, 0)

        n0 = pl.multiple_of(nbase + s_i * SUBC, SUBC)
        pltpu.sync_copy(out_t, out_hbm.at[b, :, pl.ds(n0, SUBC)])
        return carry
    lax.fori_loop(0, NSUBCH, sub_body, 0)


def kernel(input, points, support_points, indices, W, bbias):
    w1 = W[:C_IN]
    w2 = W[C_IN:]
    TN = 2048

    zflat = pl.pallas_call(
        _z_body,
        grid=(B, N // TN),
        in_specs=[
            pl.BlockSpec((1, C_IN, TN), lambda b, t: (b, 0, t)),
            pl.BlockSpec((1, 3, TN), lambda b, t: (b, 0, t)),
            pl.BlockSpec((C_IN, C_OUT), lambda b, t: (0, 0)),
            pl.BlockSpec((3, C_OUT), lambda b, t: (0, 0)),
        ],
        out_specs=pl.BlockSpec((TN, CW),
                               lambda b, t: (b * (N // TN) + t, 0)),
        out_shape=jax.ShapeDtypeStruct((B * N, CW), jnp.int32),
    )(input, points, w1, w2)

    sflat = pl.pallas_call(
        _s_body,
        grid=(B,),
        in_specs=[
            pl.BlockSpec((1, 3, NS), lambda b: (b, 0, 0)),
            pl.BlockSpec((3, C_OUT), lambda b: (0, 0)),
            pl.BlockSpec((1, C_OUT), lambda b: (0, 0)),
        ],
        out_specs=pl.BlockSpec((NS, CW), lambda b: (b, 0)),
        out_shape=jax.ShapeDtypeStruct((B * NS, CW), jnp.int32),
    )(support_points, w2, bbias.reshape(1, C_OUT))

    idx_flat = indices.astype(jnp.int32).reshape(-1)

    mesh = plsc.VectorSubcoreMesh(core_axis_name="c", subcore_axis_name="s",
                                  num_cores=NC, num_subcores=NSUB)
    out = pl.kernel(
        _sc_body,
        out_type=jax.ShapeDtypeStruct((B, C_OUT, NS), jnp.float32),
        mesh=mesh,
        compiler_params=pltpu.CompilerParams(use_tc_tiling_on_sc=False,
                                             needs_layout_passes=False),
        scratch_types=[
            pltpu.VMEM((SUBC * K,), jnp.int32),
            pltpu.VMEM((SUBC, CW), jnp.int32),
            pltpu.VMEM((GRP * K, CW), jnp.int32),
            pltpu.VMEM((GRP * K, CW), jnp.int32),
            pltpu.VMEM((C_OUT, SUBC), jnp.float32),
            pltpu.SemaphoreType.DMA,
            pltpu.SemaphoreType.DMA,
        ],
    )(zflat, idx_flat, sflat)

    return (out, support_points, indices)


# speedup vs baseline: 1.3628x; 1.0944x over previous
import functools

import jax
import jax.numpy as jnp
from jax import lax
from jax.experimental import pallas as pl
from jax.experimental.pallas import tpu as pltpu
from jax.experimental.pallas import tpu_sc as plsc

B, C_IN, N = 8, 64, 16384
NS, K = 4096, 16
C_OUT = 128
L = 16

NC, NSUB = 2, 16
NW = NC * NSUB
CH = (B * NS) // NW
WPB = NW // B
SUBC = 256
NSUBCH = CH // SUBC
GRP = 8
NGRP = SUBC // GRP
NBUF = 2
CW = C_OUT // 2


def _pack_bf16_halves(z):
    zi = lax.bitcast_convert_type(z, jnp.int32)
    rb = (zi + 0x7FFF + ((zi >> 16) & 1)) >> 16
    packed = (rb[:, :CW] & 0xFFFF) | (rb[:, CW:] << 16)
    return jnp.concatenate([packed, jnp.zeros_like(packed)], axis=1)


def _z_body(x_ref, p_ref, w1_ref, w2_ref, o_ref):
    f = jnp.concatenate([x_ref[0], p_ref[0]], axis=0)
    w = jnp.concatenate([w1_ref[...], w2_ref[...]], axis=0)
    z = lax.dot_general(f, w, (((0,), (0,)), ((), ())),
                        preferred_element_type=jnp.float32)
    o_ref[...] = _pack_bf16_halves(z)


def _s_body(sp_ref, w2_ref, b_ref, o_ref):
    s = lax.dot_general(sp_ref[0], w2_ref[...], (((0,), (0,)), ((), ())),
                        preferred_element_type=jnp.float32)
    o_ref[...] = _pack_bf16_halves(s - b_ref[...])


def _sc_body(z_hbm, idx_hbm, s_hbm, out_hbm,
             idx_v, s_v, b0, b1, out_t, m0, m1):
    bufs = (b0, b1)
    sems = (m0, m1)
    cid = lax.axis_index("c")
    sid = lax.axis_index("s")
    wid = sid * NC + cid
    b = wid // WPB
    nbase = (wid % WPB) * CH

    def start_gather(g, buf, sem):
        off = pl.multiple_of(g * (GRP * K), GRP * K)
        pltpu.make_async_copy(
            z_hbm.at[idx_v.at[pl.ds(off, GRP * K)]], buf, sem).start()

    def wait_gather(g, buf, sem):
        off = pl.multiple_of(g * (GRP * K), GRP * K)
        pltpu.make_async_copy(
            z_hbm.at[idx_v.at[pl.ds(off, GRP * K)]], buf, sem).wait()

    def compute_group(g, buf):
        pp0 = g * GRP
        iota = lax.iota(jnp.int32, L)
        himask = jnp.full((L,), -65536, jnp.int32)
        for p in range(GRP):
            pp = pp0 + p
            cols = jnp.full((L,), pp, jnp.int32)
            for q in range(CW // L):
                sl = pl.ds(L * q, L)
                acc = plsc.bitcast(buf[K * p, sl], jnp.bfloat16)
                for r in range(1, K):
                    acc = jnp.maximum(
                        acc, plsc.bitcast(buf[K * p + r, sl], jnp.bfloat16))
                mi = plsc.bitcast(acc, jnp.int32)
                lo = plsc.bitcast(mi << 16, jnp.float32)
                hi = plsc.bitcast(mi & himask, jnp.float32)
                svi = s_v[pp, sl]
                s_lo = plsc.bitcast(svi << 16, jnp.float32)
                s_hi = plsc.bitcast(svi & himask, jnp.float32)
                rows_lo = iota + (L * q)
                plsc.store_scatter(out_t, [rows_lo, cols], lo - s_lo)
                plsc.store_scatter(out_t, [rows_lo + CW, cols], hi - s_hi)

    def sub_body(s_i, carry):
        base = pl.multiple_of(wid * CH + s_i * SUBC, SUBC)
        pltpu.sync_copy(idx_hbm.at[pl.ds(base * K, SUBC * K)], idx_v)
        pltpu.sync_copy(s_hbm.at[pl.ds(base, SUBC)], s_v)

        boff = b * N

        def shift_body(i, c):
            sl = pl.ds(pl.multiple_of(i * L, L), L)
            idx_v[sl] = idx_v[sl] + boff
            return c
        lax.fori_loop(0, (SUBC * K) // L, shift_body, 0)

        for j in range(NBUF):
            start_gather(j, bufs[j], sems[j])

        def grp_body(i, c):
            g0 = NBUF * i
            for j in range(NBUF):
                g = g0 + j
                wait_gather(g, bufs[j], sems[j])
                compute_group(g, bufs[j])

                @pl.when(g + NBUF < NGRP)
                def _():
                    start_gather(g + NBUF, bufs[j], sems[j])
            return c
        lax.fori_loop(0, NGRP // NBUF, grp_body, 0)

        n0 = pl.multiple_of(nbase + s_i * SUBC, SUBC)
        pltpu.sync_copy(out_t, out_hbm.at[b, :, pl.ds(n0, SUBC)])
        return carry
    lax.fori_loop(0, NSUBCH, sub_body, 0)


def kernel(input, points, support_points, indices, W, bbias):
    w1 = W[:C_IN]
    w2 = W[C_IN:]
    TN = 2048

    zflat = pl.pallas_call(
        _z_body,
        grid=(B, N // TN),
        in_specs=[
            pl.BlockSpec((1, C_IN, TN), lambda b, t: (b, 0, t)),
            pl.BlockSpec((1, 3, TN), lambda b, t: (b, 0, t)),
            pl.BlockSpec((C_IN, C_OUT), lambda b, t: (0, 0)),
            pl.BlockSpec((3, C_OUT), lambda b, t: (0, 0)),
        ],
        out_specs=pl.BlockSpec((TN, C_OUT),
                               lambda b, t: (b * (N // TN) + t, 0)),
        out_shape=jax.ShapeDtypeStruct((B * N, C_OUT), jnp.int32),
    )(input, points, w1, w2)

    sflat = pl.pallas_call(
        _s_body,
        grid=(B,),
        in_specs=[
            pl.BlockSpec((1, 3, NS), lambda b: (b, 0, 0)),
            pl.BlockSpec((3, C_OUT), lambda b: (0, 0)),
            pl.BlockSpec((1, C_OUT), lambda b: (0, 0)),
        ],
        out_specs=pl.BlockSpec((NS, C_OUT), lambda b: (b, 0)),
        out_shape=jax.ShapeDtypeStruct((B * NS, C_OUT), jnp.int32),
    )(support_points, w2, bbias.reshape(1, C_OUT))

    idx_flat = indices.astype(jnp.int32).reshape(-1)

    mesh = plsc.VectorSubcoreMesh(core_axis_name="c", subcore_axis_name="s",
                                  num_cores=NC, num_subcores=NSUB)
    out = pl.kernel(
        _sc_body,
        out_type=jax.ShapeDtypeStruct((B, C_OUT, NS), jnp.float32),
        mesh=mesh,
        compiler_params=pltpu.CompilerParams(use_tc_tiling_on_sc=False,
                                             needs_layout_passes=False),
        scratch_types=[
            pltpu.VMEM((SUBC * K,), jnp.int32),
            pltpu.VMEM((SUBC, C_OUT), jnp.int32),
            pltpu.VMEM((GRP * K, C_OUT), jnp.int32),
            pltpu.VMEM((GRP * K, C_OUT), jnp.int32),
            pltpu.VMEM((C_OUT, SUBC), jnp.float32),
            pltpu.SemaphoreType.DMA,
            pltpu.SemaphoreType.DMA,
        ],
    )(zflat, idx_flat, sflat)

    return (out, support_points, indices)


# speedup vs baseline: 1.3704x; 1.0055x over previous
import functools

import jax
import jax.numpy as jnp
from jax import lax
from jax.experimental import pallas as pl
from jax.experimental.pallas import tpu as pltpu
from jax.experimental.pallas import tpu_sc as plsc

B, C_IN, N = 8, 64, 16384
NS, K = 4096, 16
C_OUT = 128
L = 16

NC, NSUB = 2, 16
NW = NC * NSUB
CH = (B * NS) // NW
WPB = NW // B
SUBC = 256
NSUBCH = CH // SUBC
GRP = 8
NGRP = SUBC // GRP
NBUF = 2
CW = C_OUT // 2


def _pack_bf16_halves(z):
    zi = lax.bitcast_convert_type(z, jnp.int32)
    rb = (zi + 0x7FFF + ((zi >> 16) & 1)) >> 16
    packed = (rb[:, :CW] & 0xFFFF) | (rb[:, CW:] << 16)
    return jnp.concatenate([packed, jnp.zeros_like(packed)], axis=1)


def _z_body(x_ref, p_ref, w1_ref, w2_ref, o_ref):
    f = jnp.concatenate([x_ref[0], p_ref[0]], axis=0)
    w = jnp.concatenate([w1_ref[...], w2_ref[...]], axis=0)
    z = lax.dot_general(f.astype(jnp.bfloat16), w.astype(jnp.bfloat16),
                        (((0,), (0,)), ((), ())),
                        preferred_element_type=jnp.float32)
    o_ref[...] = _pack_bf16_halves(z)


def _s_body(sp_ref, w2_ref, b_ref, o_ref):
    s = lax.dot_general(sp_ref[0], w2_ref[...], (((0,), (0,)), ((), ())),
                        preferred_element_type=jnp.float32)
    o_ref[...] = _pack_bf16_halves(s - b_ref[...])


def _sc_body(z_hbm, idx_hbm, s_hbm, out_hbm,
             idx_v, s_v, b0, b1, out_t, m0, m1):
    bufs = (b0, b1)
    sems = (m0, m1)
    cid = lax.axis_index("c")
    sid = lax.axis_index("s")
    wid = sid * NC + cid
    b = wid // WPB
    nbase = (wid % WPB) * CH

    def start_gather(g, buf, sem):
        off = pl.multiple_of(g * (GRP * K), GRP * K)
        pltpu.make_async_copy(
            z_hbm.at[idx_v.at[pl.ds(off, GRP * K)]], buf, sem).start()

    def wait_gather(g, buf, sem):
        off = pl.multiple_of(g * (GRP * K), GRP * K)
        pltpu.make_async_copy(
            z_hbm.at[idx_v.at[pl.ds(off, GRP * K)]], buf, sem).wait()

    def compute_group(g, buf):
        pp0 = g * GRP
        iota = lax.iota(jnp.int32, L)
        himask = jnp.full((L,), -65536, jnp.int32)
        for p in range(GRP):
            pp = pp0 + p
            cols = jnp.full((L,), pp, jnp.int32)
            for q in range(CW // L):
                sl = pl.ds(L * q, L)
                acc = plsc.bitcast(buf[K * p, sl], jnp.bfloat16)
                for r in range(1, K):
                    acc = jnp.maximum(
                        acc, plsc.bitcast(buf[K * p + r, sl], jnp.bfloat16))
                mi = plsc.bitcast(acc, jnp.int32)
                lo = plsc.bitcast(mi << 16, jnp.float32)
                hi = plsc.bitcast(mi & himask, jnp.float32)
                svi = s_v[pp, sl]
                s_lo = plsc.bitcast(svi << 16, jnp.float32)
                s_hi = plsc.bitcast(svi & himask, jnp.float32)
                rows_lo = iota + (L * q)
                plsc.store_scatter(out_t, [rows_lo, cols], lo - s_lo)
                plsc.store_scatter(out_t, [rows_lo + CW, cols], hi - s_hi)

    def sub_body(s_i, carry):
        base = pl.multiple_of(wid * CH + s_i * SUBC, SUBC)
        pltpu.sync_copy(idx_hbm.at[pl.ds(base * K, SUBC * K)], idx_v)
        pltpu.sync_copy(s_hbm.at[pl.ds(base, SUBC)], s_v)

        boff = b * N

        def shift_body(i, c):
            sl = pl.ds(pl.multiple_of(i * L, L), L)
            idx_v[sl] = idx_v[sl] + boff
            return c
        lax.fori_loop(0, (SUBC * K) // L, shift_body, 0)

        for j in range(NBUF):
            start_gather(j, bufs[j], sems[j])

        def grp_body(i, c):
            g0 = NBUF * i
            for j in range(NBUF):
                g = g0 + j
                wait_gather(g, bufs[j], sems[j])
                compute_group(g, bufs[j])

                @pl.when(g + NBUF < NGRP)
                def _():
                    start_gather(g + NBUF, bufs[j], sems[j])
            return c
        lax.fori_loop(0, NGRP // NBUF, grp_body, 0)

        n0 = pl.multiple_of(nbase + s_i * SUBC, SUBC)
        pltpu.sync_copy(out_t, out_hbm.at[b, :, pl.ds(n0, SUBC)])
        return carry
    lax.fori_loop(0, NSUBCH, sub_body, 0)


def kernel(input, points, support_points, indices, W, bbias):
    w1 = W[:C_IN]
    w2 = W[C_IN:]
    TN = 2048

    zflat = pl.pallas_call(
        _z_body,
        grid=(B, N // TN),
        in_specs=[
            pl.BlockSpec((1, C_IN, TN), lambda b, t: (b, 0, t)),
            pl.BlockSpec((1, 3, TN), lambda b, t: (b, 0, t)),
            pl.BlockSpec((C_IN, C_OUT), lambda b, t: (0, 0)),
            pl.BlockSpec((3, C_OUT), lambda b, t: (0, 0)),
        ],
        out_specs=pl.BlockSpec((TN, C_OUT),
                               lambda b, t: (b * (N // TN) + t, 0)),
        out_shape=jax.ShapeDtypeStruct((B * N, C_OUT), jnp.int32),
    )(input, points, w1, w2)

    sflat = pl.pallas_call(
        _s_body,
        grid=(B,),
        in_specs=[
            pl.BlockSpec((1, 3, NS), lambda b: (b, 0, 0)),
            pl.BlockSpec((3, C_OUT), lambda b: (0, 0)),
            pl.BlockSpec((1, C_OUT), lambda b: (0, 0)),
        ],
        out_specs=pl.BlockSpec((NS, C_OUT), lambda b: (b, 0)),
        out_shape=jax.ShapeDtypeStruct((B * NS, C_OUT), jnp.int32),
    )(support_points, w2, bbias.reshape(1, C_OUT))

    idx_flat = indices.astype(jnp.int32).reshape(-1)

    mesh = plsc.VectorSubcoreMesh(core_axis_name="c", subcore_axis_name="s",
                                  num_cores=NC, num_subcores=NSUB)
    out = pl.kernel(
        _sc_body,
        out_type=jax.ShapeDtypeStruct((B, C_OUT, NS), jnp.float32),
        mesh=mesh,
        compiler_params=pltpu.CompilerParams(use_tc_tiling_on_sc=False,
                                             needs_layout_passes=False),
        scratch_types=[
            pltpu.VMEM((SUBC * K,), jnp.int32),
            pltpu.VMEM((SUBC, C_OUT), jnp.int32),
            pltpu.VMEM((GRP * K, C_OUT), jnp.int32),
            pltpu.VMEM((GRP * K, C_OUT), jnp.int32),
            pltpu.VMEM((C_OUT, SUBC), jnp.float32),
            pltpu.SemaphoreType.DMA,
            pltpu.SemaphoreType.DMA,
        ],
    )(zflat, idx_flat, sflat)

    return (out, support_points, indices)


# speedup vs baseline: 1.3792x; 1.0064x over previous
import jax
import jax.numpy as jnp
from jax import lax
from jax.experimental import pallas as pl
from jax.experimental.pallas import tpu as pltpu
from jax.experimental.pallas import tpu_sc as plsc

B, C_IN, N = 8, 64, 16384
NS, K = 4096, 16
C_OUT = 128
L = 16

NC, NSUB = 2, 16
NW = NC * NSUB
CH = (B * NS) // NW
WPB = NW // B
SUBC = 256
NSUBCH = CH // SUBC
GRP = 8
NGRP = SUBC // GRP
NBUF = 2
CW = C_OUT // 2


def _pack_bf16_halves(z):
    zi = lax.bitcast_convert_type(z, jnp.int32)
    rb = (zi + 0x7FFF + ((zi >> 16) & 1)) >> 16
    packed = (rb[:, :CW] & 0xFFFF) | (rb[:, CW:] << 16)
    return jnp.concatenate([packed, jnp.zeros_like(packed)], axis=1)


def _z_body(x_ref, p_ref, w1_ref, w2_ref, o_ref):
    f = jnp.concatenate([x_ref[0], p_ref[0]], axis=0)
    w = jnp.concatenate([w1_ref[...], w2_ref[...]], axis=0)
    z = lax.dot_general(f.astype(jnp.bfloat16), w.astype(jnp.bfloat16),
                        (((0,), (0,)), ((), ())),
                        preferred_element_type=jnp.float32)
    o_ref[...] = _pack_bf16_halves(z)


def _s_body(sp_ref, w2_ref, b_ref, o_ref):
    s = lax.dot_general(sp_ref[0], w2_ref[...], (((0,), (0,)), ((), ())),
                        preferred_element_type=jnp.float32)
    o_ref[...] = _pack_bf16_halves(s - b_ref[...])


def _sc_body(z_hbm, idx_hbm, s_hbm, out_hbm,
             idx_v, s_v, b0, b1, out_t, m0, m1):
    bufs = (b0, b1)
    sems = (m0, m1)
    cid = lax.axis_index("c")
    sid = lax.axis_index("s")
    wid = sid * NC + cid
    b = wid // WPB
    nbase = (wid % WPB) * CH

    def start_gather(g, buf, sem):
        off = pl.multiple_of(g * (GRP * K), GRP * K)
        pltpu.make_async_copy(
            z_hbm.at[idx_v.at[pl.ds(off, GRP * K)]], buf, sem).start()

    def wait_gather(g, buf, sem):
        off = pl.multiple_of(g * (GRP * K), GRP * K)
        pltpu.make_async_copy(
            z_hbm.at[idx_v.at[pl.ds(off, GRP * K)]], buf, sem).wait()

    def compute_group(g, buf):
        pp0 = g * GRP
        iota = lax.iota(jnp.int32, L)
        himask = jnp.full((L,), -65536, jnp.int32)
        for p in range(GRP):
            pp = pp0 + p
            cols = jnp.full((L,), pp, jnp.int32)
            for q in range(CW // L):
                sl = pl.ds(L * q, L)
                acc = plsc.bitcast(buf[K * p, sl], jnp.bfloat16)
                for r in range(1, K):
                    acc = jnp.maximum(
                        acc, plsc.bitcast(buf[K * p + r, sl], jnp.bfloat16))
                mi = plsc.bitcast(acc, jnp.int32)
                lo = plsc.bitcast(mi << 16, jnp.float32)
                hi = plsc.bitcast(mi & himask, jnp.float32)
                svi = s_v[pp, sl]
                s_lo = plsc.bitcast(svi << 16, jnp.float32)
                s_hi = plsc.bitcast(svi & himask, jnp.float32)
                rows_lo = iota + (L * q)
                plsc.store_scatter(out_t, [rows_lo, cols], lo - s_lo)
                plsc.store_scatter(out_t, [rows_lo + CW, cols], hi - s_hi)

    def sub_body(s_i, carry):
        base = pl.multiple_of(wid * CH + s_i * SUBC, SUBC)
        pltpu.sync_copy(idx_hbm.at[pl.ds(base * K, SUBC * K)], idx_v)
        pltpu.sync_copy(s_hbm.at[pl.ds(base, SUBC)], s_v)

        boff = b * N

        def shift_body(i, c):
            sl = pl.ds(pl.multiple_of(i * L, L), L)
            idx_v[sl] = idx_v[sl] + boff
            return c
        lax.fori_loop(0, (SUBC * K) // L, shift_body, 0)

        for j in range(NBUF):
            start_gather(j, bufs[j], sems[j])

        def grp_body(i, c):
            g0 = NBUF * i
            for j in range(NBUF):
                g = g0 + j
                wait_gather(g, bufs[j], sems[j])
                compute_group(g, bufs[j])

                @pl.when(g + NBUF < NGRP)
                def _():
                    start_gather(g + NBUF, bufs[j], sems[j])
            return c
        lax.fori_loop(0, NGRP // NBUF, grp_body, 0)

        n0 = pl.multiple_of(nbase + s_i * SUBC, SUBC)
        pltpu.sync_copy(out_t, out_hbm.at[b, :, pl.ds(n0, SUBC)])
        return carry
    lax.fori_loop(0, NSUBCH, sub_body, 0)


def kernel(input, points, support_points, indices, W, bbias):
    w1 = W[:C_IN]
    w2 = W[C_IN:]
    TN = 2048

    zflat = pl.pallas_call(
        _z_body,
        grid=(B, N // TN),
        in_specs=[
            pl.BlockSpec((1, C_IN, TN), lambda b, t: (b, 0, t)),
            pl.BlockSpec((1, 3, TN), lambda b, t: (b, 0, t)),
            pl.BlockSpec((C_IN, C_OUT), lambda b, t: (0, 0)),
            pl.BlockSpec((3, C_OUT), lambda b, t: (0, 0)),
        ],
        out_specs=pl.BlockSpec((TN, C_OUT),
                               lambda b, t: (b * (N // TN) + t, 0)),
        out_shape=jax.ShapeDtypeStruct((B * N, C_OUT), jnp.int32),
    )(input, points, w1, w2)

    sflat = pl.pallas_call(
        _s_body,
        grid=(B,),
        in_specs=[
            pl.BlockSpec((1, 3, NS), lambda b: (b, 0, 0)),
            pl.BlockSpec((3, C_OUT), lambda b: (0, 0)),
            pl.BlockSpec((1, C_OUT), lambda b: (0, 0)),
        ],
        out_specs=pl.BlockSpec((NS, C_OUT), lambda b: (b, 0)),
        out_shape=jax.ShapeDtypeStruct((B * NS, C_OUT), jnp.int32),
    )(support_points, w2, bbias.reshape(1, C_OUT))

    idx_flat = indices.astype(jnp.int32).reshape(-1)

    mesh = plsc.VectorSubcoreMesh(core_axis_name="c", subcore_axis_name="s",
                                  num_cores=NC, num_subcores=NSUB)
    out = pl.kernel(
        _sc_body,
        out_type=jax.ShapeDtypeStruct((B, C_OUT, NS), jnp.float32),
        mesh=mesh,
        compiler_params=pltpu.CompilerParams(use_tc_tiling_on_sc=False,
                                             needs_layout_passes=False),
        scratch_types=[
            pltpu.VMEM((SUBC * K,), jnp.int32),
            pltpu.VMEM((SUBC, C_OUT), jnp.int32),
            pltpu.VMEM((GRP * K, C_OUT), jnp.int32),
            pltpu.VMEM((GRP * K, C_OUT), jnp.int32),
            pltpu.VMEM((C_OUT, SUBC), jnp.float32),
            pltpu.SemaphoreType.DMA,
            pltpu.SemaphoreType.DMA,
        ],
    )(zflat, idx_flat, sflat)

    return (out, support_points, indices)
